# Initial kernel scaffold; baseline (speedup 1.0000x reference)
import jax
import jax.numpy as jnp
from jax.experimental import pallas as pl


def kernel(hidden_states, router_w, gate_up_w, down_w):
    raise NotImplementedError("write your pallas kernel here")


# speedup vs baseline: 2.0056x; 2.0056x over previous
import functools

import jax
import jax.numpy as jnp
from jax import lax
from jax.experimental import pallas as pl
from jax.experimental.pallas import tpu as pltpu


def _moe_body(x_ref, rw_ref, gate_ref, up_ref, down_ref,
              out_ref, loss_ref, wcol_ref, xb_ref,
              *, M, E, TOPK, MAXD, EBF, DBF, CBF):
    e = pl.program_id(0)
    n = pl.program_id(1)

    @pl.when((e == 0) & (n == 0))
    def _prologue():
        x = x_ref[...]
        logits = lax.dot_general(
            x, rw_ref[...], (((1,), (1,)), ((), ())),
            preferred_element_type=jnp.float32)
        mx = jnp.max(logits, axis=1, keepdims=True)
        ex = jnp.exp(logits - mx)
        probs = ex / jnp.sum(ex, axis=1, keepdims=True)
        lane = lax.broadcasted_iota(jnp.int32, probs.shape, 1)
        m1 = jnp.max(probs, axis=1, keepdims=True)
        i1 = jnp.min(jnp.where(probs == m1, lane, E), axis=1, keepdims=True)
        mask1 = lane == i1
        probsb = jnp.where(mask1, -jnp.inf, probs)
        m2 = jnp.max(probsb, axis=1, keepdims=True)
        i2 = jnp.min(jnp.where(probsb == m2, lane, E), axis=1, keepdims=True)
        mask2 = lane == i2
        a = jnp.exp(m1)
        b = jnp.exp(m2)
        w1 = a / (a + b)
        w2 = b / (a + b)
        wcol_ref[...] = jnp.where(mask1, w1, 0.0) + jnp.where(mask2, w2, 0.0)
        counts = jnp.sum(mask1.astype(jnp.float32) + mask2.astype(jnp.float32),
                         axis=0, keepdims=True)
        P_i = jnp.sum(probs, axis=0, keepdims=True) / M
        f_i = counts / (M * TOPK + 1e-10)
        s1 = jnp.sum(f_i * P_i)
        eb = jnp.minimum(s1 * EBF, 10.0)
        db = jnp.minimum(s1 * DBF, 10.0)
        f_comm = counts / (M * MAXD + 1e-10)
        cb = jnp.minimum(jnp.sum(f_comm * P_i) * CBF, 10.0)
        lv = lax.broadcasted_iota(jnp.int32, (1, E), 1)
        loss_ref[...] = (jnp.where(lv == 0, eb, 0.0)
                         + jnp.where(lv == 1, db, 0.0)
                         + jnp.where(lv == 2, cb, 0.0)
                         + jnp.where(lv == 3, eb + db + cb, 0.0))
        xb_ref[...] = x.astype(jnp.bfloat16)
        out_ref[...] = jnp.zeros_like(out_ref)

    xb = xb_ref[...]
    g = gate_ref[0].astype(jnp.bfloat16)
    u = up_ref[0].astype(jnp.bfloat16)
    d = down_ref[0].astype(jnp.bfloat16)
    gu = lax.dot_general(xb, g, (((1,), (0,)), ((), ())),
                         preferred_element_type=jnp.float32)
    uu = lax.dot_general(xb, u, (((1,), (0,)), ((), ())),
                         preferred_element_type=jnp.float32)
    h = (gu / (1.0 + jnp.exp(-gu))) * uu
    lane = lax.broadcasted_iota(jnp.int32, (M, E), 1)
    w = jnp.sum(jnp.where(lane == e, wcol_ref[...], 0.0), axis=1,
                keepdims=True)
    hb = (h * w).astype(jnp.bfloat16)
    out_ref[...] += lax.dot_general(hb, d, (((1,), (0,)), ((), ())),
                                    preferred_element_type=jnp.float32)


def _moe_call(hidden_states, router_w, gate_up_w, down_w, interpret=False):
    B, S, H = hidden_states.shape
    E = router_w.shape[0]
    I = down_w.shape[1]
    M = B * S
    TN = 512 if I % 512 == 0 else I
    N = I // TN
    TOPK, MAXD = 2, 3
    EBF, DBF, CBF = 0.003, 0.05, 0.02

    x = hidden_states.reshape(M, H)
    body = functools.partial(_moe_body, M=M, E=E, TOPK=TOPK, MAXD=MAXD,
                             EBF=EBF, DBF=DBF, CBF=CBF)
    out, losses = pl.pallas_call(
        body,
        grid=(E, N),
        in_specs=[
            pl.BlockSpec((M, H), lambda e, n: (0, 0)),
            pl.BlockSpec((E, H), lambda e, n: (0, 0)),
            pl.BlockSpec((1, H, TN), lambda e, n: (e, 0, n)),
            pl.BlockSpec((1, H, TN), lambda e, n: (e, 0, n + N)),
            pl.BlockSpec((1, TN, H), lambda e, n: (e, n, 0)),
        ],
        out_specs=[
            pl.BlockSpec((M, H), lambda e, n: (0, 0)),
            pl.BlockSpec((1, E), lambda e, n: (0, 0)),
        ],
        out_shape=[
            jax.ShapeDtypeStruct((M, H), jnp.float32),
            jax.ShapeDtypeStruct((1, E), jnp.float32),
        ],
        scratch_shapes=[
            pltpu.VMEM((M, E), jnp.float32),
            pltpu.VMEM((M, H), jnp.bfloat16),
        ],
        interpret=interpret,
    )(x, router_w, gate_up_w, gate_up_w, down_w)

    output = out.reshape(B, S, H)
    eb = losses[0, 0]
    db = losses[0, 1]
    cb = losses[0, 2]
    tot = losses[0, 3]
    return output, eb, db, cb, tot


def kernel(hidden_states, router_w, gate_up_w, down_w):
    return _moe_call(hidden_states, router_w, gate_up_w, down_w)
